# baseline (speedup 1.0000x reference)
import functools

import jax
import jax.numpy as jnp
from jax import lax
from jax.experimental import pallas as pl
from jax.experimental.pallas import tpu as pltpu
from jax.experimental.pallas import tpu_sc as plsc

N = 10000
E = 320000
D = 128
NC = 2
NS = 16
NW = NC * NS
EC = 128
NP = 10240
AR = NP // NS
REP = 2560
REPH = 5120

F32 = jnp.float32


def _mesh():
    return plsc.VectorSubcoreMesh(core_axis_name="c", subcore_axis_name="s",
                                  num_cores=NC, num_subcores=NS)


CHI = 40

SLOW_T = 0
FAST_T = 160


def _sc_segment_sum(values, idx_in, idx_out, zeros_rows):
    @functools.partial(
        pl.kernel,
        out_type=jax.ShapeDtypeStruct((NC, NP, D), F32),
        mesh=_mesh(),
        scratch_types=[
            pltpu.VMEM((CHI, EC), jnp.int32),
            pltpu.VMEM((CHI, EC), jnp.int32),
            pltpu.VMEM((2, EC, D), F32),
            pltpu.VMEM_SHARED((NP, D), F32),
            pltpu.SemaphoreType.DMA,
            pltpu.SemaphoreType.DMA,
        ],
    )
    def body(vals_hbm, iin_hbm, iout_hbm, z_hbm, out_hbm,
             iin_v, iout_v, rows_v, acc, sem, sem2):
        c = lax.axis_index("c")
        s = lax.axis_index("s")
        slow = c == 0
        nch = jnp.where(slow, SLOW_T // CHI, FAST_T // CHI)
        rbase = jnp.where(slow, NS * FAST_T + s * SLOW_T, s * FAST_T)
        zoff = pl.multiple_of(s * AR, 8)
        pltpu.sync_copy(z_hbm, acc.at[pl.ds(zoff, AR)])
        plsc.subcore_barrier()

        def chunk(ci, carry):
            eoff = pl.multiple_of(rbase + ci * CHI, 8)
            pltpu.sync_copy(iin_hbm.at[pl.ds(eoff, CHI)], iin_v)
            pltpu.sync_copy(iout_hbm.at[pl.ds(eoff, CHI)], iout_v)
            pltpu.async_copy(vals_hbm.at[iin_v.at[0]], rows_v.at[0], sem)
            pltpu.async_copy(vals_hbm.at[iin_v.at[1]], rows_v.at[1], sem2)

            def pair(t, carry2):
                pltpu.make_async_copy(vals_hbm.at[iin_v.at[0]],
                                      rows_v.at[0], sem).wait()
                pltpu.sync_copy(rows_v.at[0], acc.at[iout_v.at[2 * t]],
                                add=True)

                @pl.when(t < CHI // 2 - 1)
                def _():
                    pltpu.async_copy(vals_hbm.at[iin_v.at[2 * t + 2]],
                                     rows_v.at[0], sem)

                pltpu.make_async_copy(vals_hbm.at[iin_v.at[1]],
                                      rows_v.at[1], sem2).wait()
                pltpu.sync_copy(rows_v.at[1], acc.at[iout_v.at[2 * t + 1]],
                                add=True)

                @pl.when(t < CHI // 2 - 1)
                def _():
                    pltpu.async_copy(vals_hbm.at[iin_v.at[2 * t + 3]],
                                     rows_v.at[1], sem2)

                return carry2

            lax.fori_loop(0, CHI // 2, pair, 0)
            return carry

        lax.fori_loop(0, nch, chunk, 0)
        plsc.subcore_barrier()
        pltpu.sync_copy(acc.at[pl.ds(zoff, AR)],
                        out_hbm.at[c, pl.ds(zoff, AR)])

    return body(values, idx_in, idx_out, zeros_rows)


RH = 2560
RPWH = RH // NW


def _sc_histogram(ones_tab, idx_out, zeros_rows):
    @functools.partial(
        pl.kernel,
        out_type=jax.ShapeDtypeStruct((NC, NP, D), F32),
        mesh=_mesh(),
        scratch_types=[
            pltpu.VMEM((CHI, EC), jnp.int32),
            pltpu.VMEM((EC, D), F32),
            pltpu.VMEM_SHARED((NP, D), F32),
        ],
    )
    def body(ones_hbm, iout_hbm, z_hbm, out_hbm, iout_v, rows_v, acc):
        c = lax.axis_index("c")
        s = lax.axis_index("s")
        w = s * NC + c
        zoff = pl.multiple_of(s * AR, 8)
        pltpu.sync_copy(z_hbm, acc.at[pl.ds(zoff, AR)])
        plsc.subcore_barrier()
        for p in range(2):
            pltpu.sync_copy(ones_hbm.at[pl.ds(p * EC, EC)], rows_v)

            def chunk(ci, carry):
                eoff = pl.multiple_of(p * RH + w * RPWH + ci * CHI, 8)
                pltpu.sync_copy(iout_hbm.at[pl.ds(eoff, CHI)], iout_v)

                def step(j, carry2):
                    pltpu.sync_copy(rows_v, acc.at[iout_v.at[j]], add=True)
                    return carry2

                lax.fori_loop(0, CHI, step, 0)
                return carry

            lax.fori_loop(0, RPWH // CHI, chunk, 0)
        plsc.subcore_barrier()
        pltpu.sync_copy(acc.at[pl.ds(zoff, AR)],
                        out_hbm.at[c, pl.ds(zoff, AR)])

    return body(ones_tab, idx_out, zeros_rows)


BLK = 1000
GRID = N // BLK


def _degs_from(hist_blk):
    indeg = jnp.sum(hist_blk[:, :, :64], axis=(0, 2)) * (1.0 / 64.0)
    outdeg = jnp.sum(hist_blk[:, :, 64:], axis=(0, 2)) * (1.0 / 64.0)
    return indeg, outdeg


def _dot(a, b):
    return jnp.dot(a, b, preferred_element_type=F32,
                   precision=lax.Precision.HIGHEST)


def _t1_body(x_ref, w1_ref, hist_ref, xw_ref, u_ref):
    xw = _dot(x_ref[...], w1_ref[...])
    indeg, _ = _degs_from(hist_ref[...])
    dinv = lax.rsqrt(indeg + 1.0)
    xw_ref[...] = xw
    u_ref[...] = xw * dinv[:, None]


def _t2_body(t1_ref, xw_ref, hist_ref, b1_ref, h_ref):
    indeg, _ = _degs_from(hist_ref[...])
    dinv = lax.rsqrt(indeg + 1.0)
    t1 = t1_ref[0] + t1_ref[1]
    h = t1 * dinv[:, None] + xw_ref[...] * (dinv * dinv)[:, None] + b1_ref[...]
    h_ref[...] = jnp.maximum(h, 0.0)


def _t3_body(ks_ref, h_ref, w2_ref, hist_ref, y_ref, v_ref):
    indeg, rs = _degs_from(hist_ref[...])
    rsinv = 1.0 / jnp.where(rs == 0.0, 1.0, rs)
    khf = (ks_ref[0] + ks_ref[1]) * rsinv[:, None]
    w2a = w2_ref[:D]
    w2d = w2_ref[D:] - w2a
    y = _dot(h_ref[...], w2a) + _dot(khf, w2d)
    dinv = lax.rsqrt(indeg + 1.0)
    y_ref[...] = y
    v_ref[...] = y * dinv[:, None]


def _t4_body(t2_ref, y_ref, hist_ref, b2_ref, out_ref):
    indeg, _ = _degs_from(hist_ref[...])
    dinv = lax.rsqrt(indeg + 1.0)
    t2 = t2_ref[0] + t2_ref[1]
    out_ref[...] = (t2 * dinv[:, None]
                    + y_ref[...] * (dinv * dinv)[:, None] + b2_ref[...])


def _rows(i):
    return (0, i, 0)


_SPEC_NODE = pl.BlockSpec((BLK, D), lambda i: (i, 0))
_SPEC_PART = pl.BlockSpec((NC, BLK, D), _rows)
_SPEC_B = pl.BlockSpec((D,), lambda i: (0,))
_SPEC_W = pl.BlockSpec((D, D), lambda i: (0, 0))
_SPEC_W2 = pl.BlockSpec((2 * D, D), lambda i: (0, 0))

_ND = jax.ShapeDtypeStruct((N, D), F32)


def _tc(body, in_specs, out_specs, out_shape):
    return pl.pallas_call(body, grid=(GRID,), in_specs=in_specs,
                          out_specs=out_specs, out_shape=out_shape)


def kernel(x, edge_index, W1, b1, W2, b2):
    src = edge_index[0]
    dst = edge_index[1]
    i32 = jnp.int32
    pad_trash = jnp.full((REP * EC - E,), NP - 1, i32)
    pad_zero = jnp.zeros((REP * EC - E,), i32)
    src_in = jnp.concatenate([src, pad_zero]).reshape(REP, EC)
    dst_out = jnp.concatenate([dst, pad_trash]).reshape(REP, EC)
    dst_in = jnp.concatenate([dst, pad_zero]).reshape(REP, EC)
    src_out = jnp.concatenate([src, pad_trash]).reshape(REP, EC)
    hpad = jnp.full((RH * EC - E,), NP - 1, i32)
    hist_out = jnp.concatenate([dst, hpad, src, hpad]).reshape(2 * RH, EC)
    half = jnp.concatenate([jnp.ones((EC, 64), F32),
                            jnp.zeros((EC, 64), F32)], axis=1)
    ones_tab = jnp.concatenate([half, 1.0 - half], axis=0)
    zeros_w = jnp.zeros((AR, D), F32)

    hist = _sc_histogram(ones_tab, hist_out, zeros_w)

    xw, u = _tc(_t1_body,
                [_SPEC_NODE, _SPEC_W, _SPEC_PART],
                (_SPEC_NODE, _SPEC_NODE), (_ND, _ND))(x, W1, hist)

    t1p = _sc_segment_sum(u, src_in, dst_out, zeros_w)

    h = _tc(_t2_body,
            [_SPEC_PART, _SPEC_NODE, _SPEC_PART, _SPEC_B],
            _SPEC_NODE, _ND)(t1p, xw, hist, b1)

    ksp = _sc_segment_sum(h, dst_in, src_out, zeros_w)

    y, v = _tc(_t3_body,
               [_SPEC_PART, _SPEC_NODE, _SPEC_W2, _SPEC_PART],
               (_SPEC_NODE, _SPEC_NODE), (_ND, _ND))(ksp, h, W2, hist)

    t2p = _sc_segment_sum(v, src_in, dst_out, zeros_w)

    out = _tc(_t4_body,
              [_SPEC_PART, _SPEC_NODE, _SPEC_PART, _SPEC_B],
              _SPEC_NODE, _ND)(t2p, y, hist, b2)
    return out


# speedup vs baseline: 1.2150x; 1.2150x over previous
import functools

import jax
import jax.numpy as jnp
from jax import lax
from jax.experimental import pallas as pl
from jax.experimental.pallas import tpu as pltpu
from jax.experimental.pallas import tpu_sc as plsc

N = 10000
E = 320000
D = 128
NC = 2
NS = 16
NW = NC * NS
EC = 128
NP = 10240
AR = NP // NS
REP = 2560
REPH = 5120

F32 = jnp.float32


def _mesh():
    return plsc.VectorSubcoreMesh(core_axis_name="c", subcore_axis_name="s",
                                  num_cores=NC, num_subcores=NS)


CHI = 40

SLOW_T = 40
FAST_T = 120


def _sc_segment_sum(values, idx_in, idx_out, zeros_rows):
    @functools.partial(
        pl.kernel,
        out_type=jax.ShapeDtypeStruct((NC, NP, D), F32),
        mesh=_mesh(),
        scratch_types=[
            pltpu.VMEM((CHI, EC), jnp.int32),
            pltpu.VMEM((CHI, EC), jnp.int32),
            pltpu.VMEM((2, EC, D), F32),
            pltpu.VMEM_SHARED((NP, D), F32),
            pltpu.SemaphoreType.DMA,
            pltpu.SemaphoreType.DMA,
        ],
    )
    def body(vals_hbm, iin_hbm, iout_hbm, z_hbm, out_hbm,
             iin_v, iout_v, rows_v, acc, sem, sem2):
        c = lax.axis_index("c")
        s = lax.axis_index("s")
        slow = c == 0
        nch = jnp.where(slow, SLOW_T // CHI, FAST_T // CHI)
        rbase = jnp.where(slow, NS * FAST_T + s * SLOW_T, s * FAST_T)
        zoff = pl.multiple_of(s * AR, 8)
        pltpu.sync_copy(z_hbm, acc.at[pl.ds(zoff, AR)])
        plsc.subcore_barrier()

        def chunk(ci, carry):
            eoff = pl.multiple_of(rbase + ci * CHI, 8)
            pltpu.sync_copy(iin_hbm.at[pl.ds(eoff, CHI)], iin_v)
            pltpu.sync_copy(iout_hbm.at[pl.ds(eoff, CHI)], iout_v)
            pltpu.async_copy(vals_hbm.at[iin_v.at[0]], rows_v.at[0], sem)
            pltpu.async_copy(vals_hbm.at[iin_v.at[1]], rows_v.at[1], sem2)

            def pair(t, carry2):
                pltpu.make_async_copy(vals_hbm.at[iin_v.at[0]],
                                      rows_v.at[0], sem).wait()
                pltpu.sync_copy(rows_v.at[0], acc.at[iout_v.at[2 * t]],
                                add=True)

                @pl.when(t < CHI // 2 - 1)
                def _():
                    pltpu.async_copy(vals_hbm.at[iin_v.at[2 * t + 2]],
                                     rows_v.at[0], sem)

                pltpu.make_async_copy(vals_hbm.at[iin_v.at[1]],
                                      rows_v.at[1], sem2).wait()
                pltpu.sync_copy(rows_v.at[1], acc.at[iout_v.at[2 * t + 1]],
                                add=True)

                @pl.when(t < CHI // 2 - 1)
                def _():
                    pltpu.async_copy(vals_hbm.at[iin_v.at[2 * t + 3]],
                                     rows_v.at[1], sem2)

                return carry2

            lax.fori_loop(0, CHI // 2, pair, 0)
            return carry

        lax.fori_loop(0, nch, chunk, 0)
        plsc.subcore_barrier()
        pltpu.sync_copy(acc.at[pl.ds(zoff, AR)],
                        out_hbm.at[c, pl.ds(zoff, AR)])

    return body(values, idx_in, idx_out, zeros_rows)


RH = 2560
RPWH = RH // NW


def _sc_histogram(ones_tab, idx_out, zeros_rows):
    @functools.partial(
        pl.kernel,
        out_type=jax.ShapeDtypeStruct((NC, NP, D), F32),
        mesh=_mesh(),
        scratch_types=[
            pltpu.VMEM((CHI, EC), jnp.int32),
            pltpu.VMEM((EC, D), F32),
            pltpu.VMEM_SHARED((NP, D), F32),
        ],
    )
    def body(ones_hbm, iout_hbm, z_hbm, out_hbm, iout_v, rows_v, acc):
        c = lax.axis_index("c")
        s = lax.axis_index("s")
        w = s * NC + c
        zoff = pl.multiple_of(s * AR, 8)
        pltpu.sync_copy(z_hbm, acc.at[pl.ds(zoff, AR)])
        plsc.subcore_barrier()
        for p in range(2):
            pltpu.sync_copy(ones_hbm.at[pl.ds(p * EC, EC)], rows_v)

            def chunk(ci, carry):
                eoff = pl.multiple_of(p * RH + w * RPWH + ci * CHI, 8)
                pltpu.sync_copy(iout_hbm.at[pl.ds(eoff, CHI)], iout_v)

                def step(j, carry2):
                    pltpu.sync_copy(rows_v, acc.at[iout_v.at[j]], add=True)
                    return carry2

                lax.fori_loop(0, CHI, step, 0)
                return carry

            lax.fori_loop(0, RPWH // CHI, chunk, 0)
        plsc.subcore_barrier()
        pltpu.sync_copy(acc.at[pl.ds(zoff, AR)],
                        out_hbm.at[c, pl.ds(zoff, AR)])

    return body(ones_tab, idx_out, zeros_rows)


BLK = 1000
GRID = N // BLK


def _degs_from(hist_blk):
    indeg = jnp.sum(hist_blk[:, :, :64], axis=(0, 2)) * (1.0 / 64.0)
    outdeg = jnp.sum(hist_blk[:, :, 64:], axis=(0, 2)) * (1.0 / 64.0)
    return indeg, outdeg


def _dot(a, b):
    return jnp.dot(a, b, preferred_element_type=F32,
                   precision=lax.Precision.HIGHEST)


def _t1_body(x_ref, w1_ref, hist_ref, xw_ref, u_ref):
    xw = _dot(x_ref[...], w1_ref[...])
    indeg, _ = _degs_from(hist_ref[...])
    dinv = lax.rsqrt(indeg + 1.0)
    xw_ref[...] = xw
    u_ref[...] = xw * dinv[:, None]


def _t2_body(t1_ref, xw_ref, hist_ref, b1_ref, h_ref):
    indeg, _ = _degs_from(hist_ref[...])
    dinv = lax.rsqrt(indeg + 1.0)
    t1 = t1_ref[0] + t1_ref[1]
    h = t1 * dinv[:, None] + xw_ref[...] * (dinv * dinv)[:, None] + b1_ref[...]
    h_ref[...] = jnp.maximum(h, 0.0)


def _t3_body(ks_ref, h_ref, w2_ref, hist_ref, y_ref, v_ref):
    indeg, rs = _degs_from(hist_ref[...])
    rsinv = 1.0 / jnp.where(rs == 0.0, 1.0, rs)
    khf = (ks_ref[0] + ks_ref[1]) * rsinv[:, None]
    w2a = w2_ref[:D]
    w2d = w2_ref[D:] - w2a
    y = _dot(h_ref[...], w2a) + _dot(khf, w2d)
    dinv = lax.rsqrt(indeg + 1.0)
    y_ref[...] = y
    v_ref[...] = y * dinv[:, None]


def _t4_body(t2_ref, y_ref, hist_ref, b2_ref, out_ref):
    indeg, _ = _degs_from(hist_ref[...])
    dinv = lax.rsqrt(indeg + 1.0)
    t2 = t2_ref[0] + t2_ref[1]
    out_ref[...] = (t2 * dinv[:, None]
                    + y_ref[...] * (dinv * dinv)[:, None] + b2_ref[...])


def _rows(i):
    return (0, i, 0)


_SPEC_NODE = pl.BlockSpec((BLK, D), lambda i: (i, 0))
_SPEC_PART = pl.BlockSpec((NC, BLK, D), _rows)
_SPEC_B = pl.BlockSpec((D,), lambda i: (0,))
_SPEC_W = pl.BlockSpec((D, D), lambda i: (0, 0))
_SPEC_W2 = pl.BlockSpec((2 * D, D), lambda i: (0, 0))

_ND = jax.ShapeDtypeStruct((N, D), F32)


def _tc(body, in_specs, out_specs, out_shape):
    return pl.pallas_call(body, grid=(GRID,), in_specs=in_specs,
                          out_specs=out_specs, out_shape=out_shape)


def kernel(x, edge_index, W1, b1, W2, b2):
    src = edge_index[0]
    dst = edge_index[1]
    i32 = jnp.int32
    pad_trash = jnp.full((REP * EC - E,), NP - 1, i32)
    pad_zero = jnp.zeros((REP * EC - E,), i32)
    src_in = jnp.concatenate([src, pad_zero]).reshape(REP, EC)
    dst_out = jnp.concatenate([dst, pad_trash]).reshape(REP, EC)
    dst_in = jnp.concatenate([dst, pad_zero]).reshape(REP, EC)
    src_out = jnp.concatenate([src, pad_trash]).reshape(REP, EC)
    hpad = jnp.full((RH * EC - E,), NP - 1, i32)
    hist_out = jnp.concatenate([dst, hpad, src, hpad]).reshape(2 * RH, EC)
    half = jnp.concatenate([jnp.ones((EC, 64), F32),
                            jnp.zeros((EC, 64), F32)], axis=1)
    ones_tab = jnp.concatenate([half, 1.0 - half], axis=0)
    zeros_w = jnp.zeros((AR, D), F32)

    hist = _sc_histogram(ones_tab, hist_out, zeros_w)

    xw, u = _tc(_t1_body,
                [_SPEC_NODE, _SPEC_W, _SPEC_PART],
                (_SPEC_NODE, _SPEC_NODE), (_ND, _ND))(x, W1, hist)

    t1p = _sc_segment_sum(u, src_in, dst_out, zeros_w)

    h = _tc(_t2_body,
            [_SPEC_PART, _SPEC_NODE, _SPEC_PART, _SPEC_B],
            _SPEC_NODE, _ND)(t1p, xw, hist, b1)

    ksp = _sc_segment_sum(h, dst_in, src_out, zeros_w)

    y, v = _tc(_t3_body,
               [_SPEC_PART, _SPEC_NODE, _SPEC_W2, _SPEC_PART],
               (_SPEC_NODE, _SPEC_NODE), (_ND, _ND))(ksp, h, W2, hist)

    t2p = _sc_segment_sum(v, src_in, dst_out, zeros_w)

    out = _tc(_t4_body,
              [_SPEC_PART, _SPEC_NODE, _SPEC_PART, _SPEC_B],
              _SPEC_NODE, _ND)(t2p, y, hist, b2)
    return out
